# Initial kernel scaffold; baseline (speedup 1.0000x reference)
import jax
import jax.numpy as jnp
from jax.experimental import pallas as pl


def kernel(var_feats, con_feats, edge_index, edge_val, params):
    raise NotImplementedError("write your pallas kernel here")


# speedup vs baseline: 1.0005x; 1.0005x over previous
import functools

import jax
import jax.numpy as jnp
from jax.experimental import pallas as pl
from jax.experimental.pallas import tpu as pltpu

H = 128
N_PROBES = 16
N_HEADS = 4
N_LAYERS = 2
CLIP = 5.0
_VAR_LOG = [19]
_VAR_STD = [0, 7, 8, 9, 12, 14, 19, 20]
_CON_LOG = [5]
_CON_STD = [0, 1, 3, 4, 5]


def _lin(x, W, b):
    return x @ W.T + b


def _ln(x, g, b):
    m = x.mean(-1, keepdims=True)
    v = ((x - m) ** 2).mean(-1, keepdims=True)
    return (x - m) / jnp.sqrt(v + 1e-5) * g + b


def _scatter_mean(src, idx, n):
    out = jax.ops.segment_sum(src, idx, num_segments=n)
    cnt = jax.ops.segment_sum(jnp.ones(idx.shape[0], src.dtype), idx, num_segments=n)
    return out / jnp.clip(cnt, 1.0, None)[:, None]


def _normalize(x, log_cols, std_cols):
    out = x
    for c in log_cols:
        out = out.at[:, c].set(jnp.log1p(jnp.abs(out[:, c])) * jnp.sign(out[:, c]))
    cols = jnp.array(std_cols)
    v = out[:, cols]
    vn = jnp.clip((v - v.mean(0)) / (v.std(0, ddof=1) + 1e-06), -CLIP, CLIP)
    return out.at[:, cols].set(vn)


def _normalize_edge(e):
    return jnp.clip((e - e.mean()) / (e.std(ddof=1) + 1e-06), -CLIP, CLIP)


def _emb(x, p, pre):
    h = jax.nn.relu(_ln(_lin(x, p[pre + 'W1'], p[pre + 'b1']), p[pre + 'g1'], p[pre + 'be1']))
    h = jax.nn.relu(_ln(_lin(h, p[pre + 'W2'], p[pre + 'b2']), p[pre + 'g2'], p[pre + 'be2']))
    return h


def _gcn_layer(vh, ch, ci, vi, e, p, pre):
    msg = _lin(vh, p[pre + 'v2c_lin_W'], p[pre + 'v2c_lin_b'])[vi] * jax.nn.sigmoid(
        _lin(e, p[pre + 'v2c_gate_W'], p[pre + 'v2c_gate_b']))
    agg = _scatter_mean(msg, ci, ch.shape[0])
    ch_new = jax.nn.relu(_ln(_lin(jnp.concatenate([agg, ch], -1), p[pre + 'v2c_upd_W'],
                                  p[pre + 'v2c_upd_b']), p[pre + 'v2c_upd_g'], p[pre + 'v2c_upd_be']))
    msg = _lin(ch_new, p[pre + 'c2v_lin_W'], p[pre + 'c2v_lin_b'])[ci] * jax.nn.sigmoid(
        _lin(e, p[pre + 'c2v_gate_W'], p[pre + 'c2v_gate_b']))
    agg = _scatter_mean(msg, vi, vh.shape[0])
    vh_new = jax.nn.relu(_ln(_lin(jnp.concatenate([agg, vh], -1), p[pre + 'c2v_upd_W'],
                                  p[pre + 'c2v_upd_b']), p[pre + 'c2v_upd_g'], p[pre + 'c2v_upd_be']))
    return vh_new, ch_new


def _mha(x, Wqkv, bqkv, Wo, bo):
    S_, d = x.shape
    qkv = x @ Wqkv.T + bqkv
    q, k, v = jnp.split(qkv, 3, axis=-1)
    dh = d // N_HEADS
    q = q.reshape(S_, N_HEADS, dh).transpose(1, 0, 2)
    k = k.reshape(S_, N_HEADS, dh).transpose(1, 0, 2)
    v = v.reshape(S_, N_HEADS, dh).transpose(1, 0, 2)
    a = jax.nn.softmax(q @ k.transpose(0, 2, 1) / jnp.sqrt(float(dh)), -1)
    o = (a @ v).transpose(1, 0, 2).reshape(S_, d)
    return o @ Wo.T + bo


def _head_body(v_ref, w_ref, b_ref, o_ref):
    o_ref[...] = jnp.sum(v_ref[...] * w_ref[...], axis=-1, keepdims=True) + b_ref[...]


def _head_pallas(V_updated, W, b):
    n = V_updated.shape[0]
    blk = 2000
    return pl.pallas_call(
        _head_body,
        grid=(n // blk,),
        in_specs=[
            pl.BlockSpec((blk, H), lambda i: (i, 0)),
            pl.BlockSpec((1, H), lambda i: (0, 0)),
            pl.BlockSpec((1, 1), lambda i: (0, 0)),
        ],
        out_specs=pl.BlockSpec((blk, 1), lambda i: (i, 0)),
        out_shape=jax.ShapeDtypeStruct((n, 1), jnp.float32),
    )(V_updated, W, b.reshape(1, 1))


@jax.jit
def kernel(var_feats, con_feats, edge_index, edge_val, params):
    p = params
    vf = _normalize(var_feats, _VAR_LOG, _VAR_STD)
    cf = _normalize(con_feats, _CON_LOG, _CON_STD)
    ew = _normalize_edge(edge_val)
    vh = _emb(vf, p, 'var_')
    ch = _emb(cf, p, 'con_')
    ci, vi = edge_index[0], edge_index[1]
    e = ew[:, None]
    for l in range(N_LAYERS):
        dv, dc = _gcn_layer(vh, ch, ci, vi, e, p, 'l%d_' % l)
        vh = vh + dv
        ch = ch + dc
    K = _lin(vh, p['Wk'], p['bk'])
    V = _lin(vh, p['Wv'], p['bv'])
    log_n = jnp.maximum(jnp.log(jnp.array(float(vh.shape[0]), jnp.float32)), 1.0)
    S = p['Q_macro'] @ K.T * log_n / jnp.sqrt(float(H))
    A_fwd = jax.nn.softmax(S, -1)
    H_macro = A_fwd @ V
    h = H_macro
    h = h + _mha(_ln(h, p['sa_g'], p['sa_b']), p['in_W'], p['in_b'], p['out_W'], p['out_b'])
    hf = _ln(h, p['ff_g'], p['ff_b'])
    hf = _lin(jax.nn.gelu(_lin(hf, p['ffn_W1'], p['ffn_b1']), approximate=False), p['ffn_W2'], p['ffn_b2'])
    H_post = h + hf
    A_bwd = jax.nn.softmax(S.T, -1)
    H_feedback = A_bwd @ _lin(H_post, p['Wvb'], p['bvb'])
    w = jnp.linalg.norm(H_macro, axis=-1)
    w = w / (w.sum() + 1e-08)
    S_i = (w[:, None] * A_fwd).sum(0)
    S_i = S_i * S_i.shape[0]
    c = jax.nn.sigmoid(p['gamma'] * S_i + p['beta'])[:, None]
    V_updated = c * H_feedback + (1.0 - c) * vh
    return _head_pallas(V_updated, p['head_W'], p['head_b'])


# speedup vs baseline: 3.1377x; 3.1362x over previous
import functools

import jax
import jax.numpy as jnp
from jax import lax
from jax.experimental import pallas as pl
from jax.experimental.pallas import tpu as pltpu
from jax.experimental.pallas import tpu_sc as plsc

_E = 320000
_N = 10000
_CHUNK = 128
_NCHUNKS = _E // _CHUNK
_NC, _NS = 2, 16
_NPAD = 10240
_ROWS_PER_TILE = _NPAD // _NS
_WSTEP = _ROWS_PER_TILE // 5


_NW = _NC * _NS
_TPW = 106
_CHK = 96
_EPAD = _NW * _TPW * _CHK
_PADROW = 10016
_WOUT = 64


def _sc_msg_body(table, packed, gwb, zacc,
                 acc_out, cnt_out,
                 rows0_v, rows1_v, idxa_v, idxb_v, gwb_v, cnt_v,
                 acc_sh, g0, g1):
    cid = lax.axis_index("c")
    sid = lax.axis_index("s")
    wid = sid * _NC + cid

    pltpu.sync_copy(gwb, gwb_v)
    pltpu.sync_copy(zacc, rows0_v)
    pltpu.sync_copy(zacc.at[pl.ds(0, 80)], cnt_v)

    base_row = sid * _ROWS_PER_TILE
    for j in range(_ROWS_PER_TILE // _WOUT):
        pltpu.sync_copy(rows0_v.at[pl.ds(0, _WOUT)],
                        acc_sh.at[pl.ds(base_row + j * _WOUT, _WOUT)])
    plsc.subcore_barrier()

    wn = [gwb_v[0, pl.ds(g * 16, 16)] for g in range(8)]
    bn = [gwb_v[1, pl.ds(g * 16, 16)] for g in range(8)]
    ones16 = jnp.ones((16,), jnp.float32)
    c127 = jnp.full((16,), 127, jnp.int32)

    def compute_and_scatter(idx_v, rows_v):
        def edge_body(i, c2):
            evi = plsc.load_gather(idx_v, [jnp.full((16,), 2, jnp.int32),
                                           jnp.full((16,), i, jnp.int32)])
            evb = plsc.bitcast(evi, jnp.float32)
            for g in range(8):
                sl = pl.ds(g * 16, 16)
                sgm = 1.0 + jnp.exp(evb * wn[g] + bn[g])
                rows_v[i, sl] = rows_v[i, sl] / sgm
            return c2

        lax.fori_loop(0, _CHK, edge_body, 0)
        pltpu.sync_copy(rows_v, acc_sh.at[idx_v.at[1]], add=True)
        for g in range(_CHK // 16):
            d16 = idx_v[1, pl.ds(g * 16, 16)]
            plsc.addupdate_scatter(
                cnt_v,
                [lax.shift_right_logical(d16, 7), lax.bitwise_and(d16, c127)],
                ones16)

    pltpu.sync_copy(packed.at[wid, 0], idxa_v)
    pltpu.async_copy(table.at[idxa_v.at[0]], rows0_v, g0)

    def pair_body(t, carry):
        a = 2 * t
        pltpu.sync_copy(packed.at[wid, a + 1], idxb_v)
        pltpu.async_copy(table.at[idxb_v.at[0]], rows1_v, g1)
        pltpu.make_async_copy(table.at[idxa_v.at[0]], rows0_v, g0).wait()
        compute_and_scatter(idxa_v, rows0_v)

        @pl.when(t < _TPW // 2 - 1)
        def _():
            pltpu.sync_copy(packed.at[wid, a + 2], idxa_v)
            pltpu.async_copy(table.at[idxa_v.at[0]], rows0_v, g0)

        pltpu.make_async_copy(table.at[idxb_v.at[0]], rows1_v, g1).wait()
        compute_and_scatter(idxb_v, rows1_v)
        return carry

    lax.fori_loop(0, _TPW // 2, pair_body, 0)
    plsc.subcore_barrier()

    for j in range(_ROWS_PER_TILE // _WOUT):
        r0 = base_row + j * _WOUT
        pltpu.sync_copy(acc_sh.at[pl.ds(r0, _WOUT)], rows0_v.at[pl.ds(0, _WOUT)])
        pltpu.sync_copy(rows0_v.at[pl.ds(0, _WOUT)], acc_out.at[cid, pl.ds(r0, _WOUT)])
    pltpu.sync_copy(cnt_v, cnt_out.at[wid])


def _sc_msg_pass(table, packed, gate_w, gate_b):
    gwb = jnp.stack([-gate_w, -gate_b])
    zacc = jnp.zeros((_CHK, H), jnp.float32)
    mesh = plsc.VectorSubcoreMesh(core_axis_name="c", subcore_axis_name="s")
    f = pl.kernel(
        _sc_msg_body,
        mesh=mesh,
        compiler_params=pltpu.CompilerParams(needs_layout_passes=False),
        out_type=(
            jax.ShapeDtypeStruct((_NC, _NPAD, H), jnp.float32),
            jax.ShapeDtypeStruct((_NW, 80, 128), jnp.float32),
        ),
        scratch_types=[
            pltpu.VMEM((_CHK, H), jnp.float32),
            pltpu.VMEM((_CHK, H), jnp.float32),
            pltpu.VMEM((3, _CHK), jnp.int32),
            pltpu.VMEM((3, _CHK), jnp.int32),
            pltpu.VMEM((2, H), jnp.float32),
            pltpu.VMEM((80, 128), jnp.float32),
            pltpu.VMEM_SHARED((_NPAD, H), jnp.float32),
            pltpu.SemaphoreType.DMA,
            pltpu.SemaphoreType.DMA,
        ],
    )
    return f(table, packed, gwb, zacc)


def _pad_edges(src, dst, ev):
    pad = _EPAD - _E
    srcp = jnp.concatenate([src, jnp.zeros((pad,), jnp.int32)])
    dstp = jnp.concatenate([dst, jnp.full((pad,), _PADROW, jnp.int32)])
    evp = jnp.concatenate([ev, jnp.zeros((pad,), jnp.float32)])
    evi = jax.lax.bitcast_convert_type(evp, jnp.int32)
    packed = jnp.stack([srcp, dstp, evi])
    packed = packed.reshape(3, _NW, _TPW, _CHK).transpose(1, 2, 0, 3)
    return packed


H = 128
N_PROBES = 16
N_HEADS = 4
N_LAYERS = 2
CLIP = 5.0
_VAR_LOG = [19]
_VAR_STD = [0, 7, 8, 9, 12, 14, 19, 20]
_CON_LOG = [5]
_CON_STD = [0, 1, 3, 4, 5]


def _lin(x, W, b):
    return x @ W.T + b


def _ln(x, g, b):
    m = x.mean(-1, keepdims=True)
    v = ((x - m) ** 2).mean(-1, keepdims=True)
    return (x - m) / jnp.sqrt(v + 1e-5) * g + b


def _scatter_mean(src, idx, n):
    out = jax.ops.segment_sum(src, idx, num_segments=n)
    cnt = jax.ops.segment_sum(jnp.ones(idx.shape[0], src.dtype), idx, num_segments=n)
    return out / jnp.clip(cnt, 1.0, None)[:, None]


def _normalize(x, log_cols, std_cols):
    out = x
    for c in log_cols:
        out = out.at[:, c].set(jnp.log1p(jnp.abs(out[:, c])) * jnp.sign(out[:, c]))
    cols = jnp.array(std_cols)
    v = out[:, cols]
    vn = jnp.clip((v - v.mean(0)) / (v.std(0, ddof=1) + 1e-06), -CLIP, CLIP)
    return out.at[:, cols].set(vn)


def _normalize_edge(e):
    return jnp.clip((e - e.mean()) / (e.std(ddof=1) + 1e-06), -CLIP, CLIP)


def _emb(x, p, pre):
    h = jax.nn.relu(_ln(_lin(x, p[pre + 'W1'], p[pre + 'b1']), p[pre + 'g1'], p[pre + 'be1']))
    h = jax.nn.relu(_ln(_lin(h, p[pre + 'W2'], p[pre + 'b2']), p[pre + 'g2'], p[pre + 'be2']))
    return h


def _sc_scatter_mean(table, packed, gate_w, gate_b, cnt):
    acc, cloc = _sc_msg_pass(table, packed, gate_w[:, 0], gate_b)
    if cnt is None:
        cnt = jnp.clip(cloc.sum(0).reshape(_NPAD)[:_N], 1.0, None)
    return (acc[0, :_N] + acc[1, :_N]) / cnt[:, None], cnt


def _gcn_layer(vh, ch, v2c3, c2v3, cnt_c, cnt_v, p, pre):
    table = _lin(vh, p[pre + 'v2c_lin_W'], p[pre + 'v2c_lin_b'])
    agg, cnt_c = _sc_scatter_mean(table, v2c3, p[pre + 'v2c_gate_W'], p[pre + 'v2c_gate_b'], cnt_c)
    ch_new = jax.nn.relu(_ln(_lin(jnp.concatenate([agg, ch], -1), p[pre + 'v2c_upd_W'],
                                  p[pre + 'v2c_upd_b']), p[pre + 'v2c_upd_g'], p[pre + 'v2c_upd_be']))
    table = _lin(ch_new, p[pre + 'c2v_lin_W'], p[pre + 'c2v_lin_b'])
    agg, cnt_v = _sc_scatter_mean(table, c2v3, p[pre + 'c2v_gate_W'], p[pre + 'c2v_gate_b'], cnt_v)
    vh_new = jax.nn.relu(_ln(_lin(jnp.concatenate([agg, vh], -1), p[pre + 'c2v_upd_W'],
                                  p[pre + 'c2v_upd_b']), p[pre + 'c2v_upd_g'], p[pre + 'c2v_upd_be']))
    return vh_new, ch_new, cnt_c, cnt_v


def _mha(x, Wqkv, bqkv, Wo, bo):
    S_, d = x.shape
    qkv = x @ Wqkv.T + bqkv
    q, k, v = jnp.split(qkv, 3, axis=-1)
    dh = d // N_HEADS
    q = q.reshape(S_, N_HEADS, dh).transpose(1, 0, 2)
    k = k.reshape(S_, N_HEADS, dh).transpose(1, 0, 2)
    v = v.reshape(S_, N_HEADS, dh).transpose(1, 0, 2)
    a = jax.nn.softmax(q @ k.transpose(0, 2, 1) / jnp.sqrt(float(dh)), -1)
    o = (a @ v).transpose(1, 0, 2).reshape(S_, d)
    return o @ Wo.T + bo


def _head_body(v_ref, w_ref, b_ref, o_ref):
    o_ref[...] = jnp.sum(v_ref[...] * w_ref[...], axis=-1, keepdims=True) + b_ref[...]


def _head_pallas(V_updated, W, b):
    n = V_updated.shape[0]
    blk = 2000
    return pl.pallas_call(
        _head_body,
        grid=(n // blk,),
        in_specs=[
            pl.BlockSpec((blk, H), lambda i: (i, 0)),
            pl.BlockSpec((1, H), lambda i: (0, 0)),
            pl.BlockSpec((1, 1), lambda i: (0, 0)),
        ],
        out_specs=pl.BlockSpec((blk, 1), lambda i: (i, 0)),
        out_shape=jax.ShapeDtypeStruct((n, 1), jnp.float32),
    )(V_updated, W, b.reshape(1, 1))


@jax.jit
def kernel(var_feats, con_feats, edge_index, edge_val, params):
    p = params
    vf = _normalize(var_feats, _VAR_LOG, _VAR_STD)
    cf = _normalize(con_feats, _CON_LOG, _CON_STD)
    ew = _normalize_edge(edge_val)
    vh = _emb(vf, p, 'var_')
    ch = _emb(cf, p, 'con_')
    ci, vi = edge_index[0], edge_index[1]
    v2c3 = _pad_edges(vi, ci, ew)
    c2v3 = _pad_edges(ci, vi, ew)
    cnt_c = cnt_v = None
    for l in range(N_LAYERS):
        dv, dc, cnt_c, cnt_v = _gcn_layer(vh, ch, v2c3, c2v3, cnt_c, cnt_v, p, 'l%d_' % l)
        vh = vh + dv
        ch = ch + dc
    K = _lin(vh, p['Wk'], p['bk'])
    V = _lin(vh, p['Wv'], p['bv'])
    log_n = jnp.maximum(jnp.log(jnp.array(float(vh.shape[0]), jnp.float32)), 1.0)
    S = p['Q_macro'] @ K.T * log_n / jnp.sqrt(float(H))
    A_fwd = jax.nn.softmax(S, -1)
    H_macro = A_fwd @ V
    h = H_macro
    h = h + _mha(_ln(h, p['sa_g'], p['sa_b']), p['in_W'], p['in_b'], p['out_W'], p['out_b'])
    hf = _ln(h, p['ff_g'], p['ff_b'])
    hf = _lin(jax.nn.gelu(_lin(hf, p['ffn_W1'], p['ffn_b1']), approximate=False), p['ffn_W2'], p['ffn_b2'])
    H_post = h + hf
    A_bwd = jax.nn.softmax(S.T, -1)
    H_feedback = A_bwd @ _lin(H_post, p['Wvb'], p['bvb'])
    w = jnp.linalg.norm(H_macro, axis=-1)
    w = w / (w.sum() + 1e-08)
    S_i = (w[:, None] * A_fwd).sum(0)
    S_i = S_i * S_i.shape[0]
    c = jax.nn.sigmoid(p['gamma'] * S_i + p['beta'])[:, None]
    V_updated = c * H_feedback + (1.0 - c) * vh
    return _head_pallas(V_updated, p['head_W'], p['head_b'])
